# Initial kernel scaffold; baseline (speedup 1.0000x reference)
import jax
import jax.numpy as jnp
from jax.experimental import pallas as pl


def kernel(history, current, poi_distance_matrix):
    raise NotImplementedError("write your pallas kernel here")


# speedup vs baseline: 1.1977x; 1.1977x over previous
import functools

import jax
import jax.numpy as jnp
from jax import lax
from jax.experimental import pallas as pl
from jax.experimental.pallas import tpu as pltpu
from jax.experimental.pallas import tpu_sc as plsc

P = 4096
SEQ = 4096
STATE = 2048
L = 16
CH = 8
U = 8
NVEC = SEQ // L


def _sc_body(hist_hbm, cur_hbm, dist_hbm, out_hbm,
             hist_v, cur_v, rows_v, e_v, sem):
    info = plsc.get_sparse_core_info()
    nc, ns = info.num_cores, info.num_subcores
    nw = nc * ns
    rows_per_w = STATE // nw
    wid = lax.axis_index("s") * nc + lax.axis_index("c")
    base = wid * rows_per_w

    pltpu.sync_copy(hist_hbm, hist_v)
    pltpu.sync_copy(cur_hbm.at[pl.ds(base, rows_per_w)], cur_v)

    iota = lax.iota(jnp.int32, L)

    def row_body(t, carry):
        c = t // CH
        k = t - c * CH
        i = base + t
        kvec = jnp.full((L,), k, jnp.int32)

        @pl.when(k == 0)
        def _():
            pltpu.async_copy(
                dist_hbm.at[cur_v.at[pl.ds(c * CH, CH)]], rows_v, sem).wait()

        def p1(vb, m_acc):
            for u in range(U):
                v = vb * U + u
                idx = hist_v[pl.ds(v * L, L)]
                g = plsc.load_gather(rows_v, [kvec, idx])
                inv = 1.0 / g
                jv = iota + v * L
                e = jnp.where(jv <= i, inv, 0.0)
                e_v[pl.ds(v * L, L)] = e
                m_acc = jnp.maximum(m_acc, e)
            return m_acc

        m_acc = lax.fori_loop(0, NVEC // U, p1,
                              jnp.zeros((L,), jnp.float32))
        m = jnp.max(m_acc)

        def p2(vb, s_acc):
            for u in range(U):
                v = vb * U + u
                e = e_v[pl.ds(v * L, L)]
                px = jnp.exp(e - m)
                e_v[pl.ds(v * L, L)] = px
                s_acc = s_acc + px
            return s_acc

        s_acc = lax.fori_loop(0, NVEC // U, p2,
                              jnp.zeros((L,), jnp.float32))
        r = 1.0 / (jnp.zeros((L,), jnp.float32) + jnp.sum(s_acc))

        def p3(vb, cc):
            for u in range(U):
                v = vb * U + u
                e_v[pl.ds(v * L, L)] = e_v[pl.ds(v * L, L)] * r
            return cc

        lax.fori_loop(0, NVEC // U, p3, 0)
        pltpu.sync_copy(e_v, out_hbm.at[i])
        return carry

    lax.fori_loop(0, rows_per_w, row_body, 0)


_sc_attn = functools.partial(
    pl.kernel,
    out_type=jax.ShapeDtypeStruct((STATE, SEQ), jnp.float32),
    mesh=plsc.VectorSubcoreMesh(core_axis_name="c", subcore_axis_name="s"),
    compiler_params=pltpu.CompilerParams(
        use_tc_tiling_on_sc=False, needs_layout_passes=False),
    scratch_types=[
        pltpu.VMEM((SEQ,), jnp.int32),
        pltpu.VMEM((STATE // 32,), jnp.int32),
        pltpu.VMEM((CH, SEQ), jnp.float32),
        pltpu.VMEM((SEQ,), jnp.float32),
        pltpu.SemaphoreType.DMA,
    ],
)(_sc_body)


def kernel(history, current, poi_distance_matrix):
    hist = history.astype(jnp.int32)
    cur = current.astype(jnp.int32)
    return _sc_attn(hist, cur, poi_distance_matrix)


# speedup vs baseline: 2.3921x; 1.9972x over previous
import functools

import jax
import jax.numpy as jnp
from jax import lax
from jax.experimental import pallas as pl
from jax.experimental.pallas import tpu as pltpu
from jax.experimental.pallas import tpu_sc as plsc

P = 4096
SEQ = 4096
STATE = 2048
L = 16
CH = 16
U = 8
NVEC = SEQ // L


def _sc_body(hist_hbm, cur_hbm, dist_hbm, out_hbm,
             hist_v, cur_all_v, idx16_v, rows_v, e_v, sem):
    info = plsc.get_sparse_core_info()
    nc, ns = info.num_cores, info.num_subcores
    nw = nc * ns
    rows_per_w = STATE // nw
    wid = lax.axis_index("s") * nc + lax.axis_index("c")

    pltpu.sync_copy(hist_hbm, hist_v)
    pltpu.sync_copy(cur_hbm, cur_all_v)

    iota = lax.iota(jnp.int32, L)
    zf = jnp.zeros((L,), jnp.float32)

    def row_body(t, carry):
        c = t >> 4
        k = t - (c << 4)
        i = wid + t * nw
        kvec = jnp.full((L,), k, jnp.int32)

        @pl.when(k == 0)
        def _():
            rowidx = plsc.load_gather(
                cur_all_v, [wid + (c * CH + iota) * nw])
            idx16_v[pl.ds(0, L)] = rowidx
            pltpu.async_copy(dist_hbm.at[idx16_v], rows_v, sem).wait()

        nv = (i >> 4) + 1
        nvb = (nv + (U - 1)) >> 3
        nv2 = nvb * U

        def p1(vb, m_acc):
            for u in range(U):
                v = vb * U + u
                idx = hist_v[pl.ds(v * L, L)]
                g = plsc.load_gather(rows_v, [kvec, idx])
                inv = 1.0 / g
                jv = iota + v * L
                e = jnp.where(jv <= i, inv, 0.0)
                e_v[pl.ds(v * L, L)] = e
                m_acc = jnp.maximum(m_acc, e)
            return m_acc

        m_acc = lax.fori_loop(0, nvb, p1, zf)
        m = jnp.max(m_acc)

        def p2(vb, s_acc):
            for u in range(U):
                v = vb * U + u
                e = e_v[pl.ds(v * L, L)]
                px = jnp.exp(e - m)
                e_v[pl.ds(v * L, L)] = px
                s_acc = s_acc + px
            return s_acc

        s_acc = lax.fori_loop(0, nvb, p2, zf)

        em = jnp.exp(zf - m)
        cntf = (SEQ - nv2 * L).astype(jnp.float32)
        s_v = (zf + jnp.sum(s_acc)) + cntf * em
        r_v = 1.0 / s_v
        tv = em * r_v

        def p3(vb, cc):
            for u in range(U):
                v = vb * U + u
                e_v[pl.ds(v * L, L)] = e_v[pl.ds(v * L, L)] * r_v
            return cc

        lax.fori_loop(0, nvb, p3, 0)

        def pfill(vb, cc):
            for u in range(U):
                v = vb * U + u
                e_v[pl.ds(v * L, L)] = tv
            return cc

        lax.fori_loop(nvb, NVEC // U, pfill, 0)
        pltpu.sync_copy(e_v, out_hbm.at[i])
        return carry

    lax.fori_loop(0, rows_per_w, row_body, 0)


_sc_attn = functools.partial(
    pl.kernel,
    out_type=jax.ShapeDtypeStruct((STATE, SEQ), jnp.float32),
    mesh=plsc.VectorSubcoreMesh(core_axis_name="c", subcore_axis_name="s"),
    compiler_params=pltpu.CompilerParams(
        use_tc_tiling_on_sc=False, needs_layout_passes=False),
    scratch_types=[
        pltpu.VMEM((SEQ,), jnp.int32),
        pltpu.VMEM((STATE,), jnp.int32),
        pltpu.VMEM((L,), jnp.int32),
        pltpu.VMEM((CH, SEQ), jnp.float32),
        pltpu.VMEM((SEQ,), jnp.float32),
        pltpu.SemaphoreType.DMA,
    ],
)(_sc_body)


def kernel(history, current, poi_distance_matrix):
    hist = history.astype(jnp.int32)
    cur = current.astype(jnp.int32)
    return _sc_attn(hist, cur, poi_distance_matrix)


# speedup vs baseline: 3.3390x; 1.3959x over previous
import functools

import jax
import jax.numpy as jnp
from jax import lax
from jax.experimental import pallas as pl
from jax.experimental.pallas import tpu as pltpu
from jax.experimental.pallas import tpu_sc as plsc

P = 4096
SEQ = 4096
STATE = 2048
L = 16
CH = 16
U = 8
NVEC = SEQ // L


def _sc_body(hist_hbm, cur_hbm, dist_hbm, out_hbm,
             hist_v, cur_all_v, idx16_v, rows_v, e_v, sem):
    info = plsc.get_sparse_core_info()
    nc, ns = info.num_cores, info.num_subcores
    nw = nc * ns
    rows_per_w = STATE // nw
    wid = lax.axis_index("s") * nc + lax.axis_index("c")

    pltpu.sync_copy(hist_hbm, hist_v)
    pltpu.sync_copy(cur_hbm, cur_all_v)

    iota = lax.iota(jnp.int32, L)
    zf = jnp.zeros((L,), jnp.float32)

    def row_body(t, carry):
        c = t >> 4
        k = t - (c << 4)
        i = wid + t * nw
        kvec = jnp.full((L,), k, jnp.int32)

        @pl.when(k == 0)
        def _():
            rowidx = plsc.load_gather(
                cur_all_v, [wid + (c * CH + iota) * nw])
            idx16_v[pl.ds(0, L)] = rowidx
            pltpu.async_copy(dist_hbm.at[idx16_v], rows_v, sem).wait()

        nfull = (i + 1) >> 4
        nfb8 = (nfull >> 3) << 3
        nv2 = nfb8 + 16

        @plsc.parallel_loop(0, nfb8, unroll=U, carry=zf)
        def m_acc(v, m_acc):
            idx = hist_v[pl.ds(v * L, L)]
            g = plsc.load_gather(rows_v, [kvec, idx])
            inv = 1.0 / g
            e_v[pl.ds(v * L, L)] = inv
            return jnp.maximum(m_acc, inv)

        @plsc.parallel_loop(nfb8, nv2, unroll=U, carry=m_acc)
        def m_acc(v, m_acc):
            idx = hist_v[pl.ds(v * L, L)]
            g = plsc.load_gather(rows_v, [kvec, idx])
            inv = 1.0 / g
            jv = iota + v * L
            e = jnp.where(jv <= i, inv, 0.0)
            e_v[pl.ds(v * L, L)] = e
            return jnp.maximum(m_acc, e)

        m = jnp.max(m_acc)

        @plsc.parallel_loop(0, nv2, unroll=U, carry=zf)
        def s_acc(v, s_acc):
            e = e_v[pl.ds(v * L, L)]
            px = jnp.exp(e - m)
            e_v[pl.ds(v * L, L)] = px
            return s_acc + px

        em = jnp.exp(zf - m)
        cntf = (SEQ - nv2 * L).astype(jnp.float32)
        s_v = (zf + jnp.sum(s_acc)) + cntf * em
        r_v = 1.0 / s_v
        tv = em * r_v

        @plsc.parallel_loop(0, nv2, unroll=U)
        def _(v):
            e_v[pl.ds(v * L, L)] = e_v[pl.ds(v * L, L)] * r_v

        @plsc.parallel_loop(nv2, NVEC, unroll=U)
        def _(v):
            e_v[pl.ds(v * L, L)] = tv
        pltpu.sync_copy(e_v, out_hbm.at[i])
        return carry

    lax.fori_loop(0, rows_per_w, row_body, 0)


_sc_attn = functools.partial(
    pl.kernel,
    out_type=jax.ShapeDtypeStruct((STATE, SEQ), jnp.float32),
    mesh=plsc.VectorSubcoreMesh(core_axis_name="c", subcore_axis_name="s"),
    compiler_params=pltpu.CompilerParams(
        use_tc_tiling_on_sc=False, needs_layout_passes=False),
    scratch_types=[
        pltpu.VMEM((SEQ,), jnp.int32),
        pltpu.VMEM((STATE,), jnp.int32),
        pltpu.VMEM((L,), jnp.int32),
        pltpu.VMEM((CH, SEQ), jnp.float32),
        pltpu.VMEM((SEQ,), jnp.float32),
        pltpu.SemaphoreType.DMA,
    ],
)(_sc_body)


def kernel(history, current, poi_distance_matrix):
    hist = history.astype(jnp.int32)
    cur = current.astype(jnp.int32)
    return _sc_attn(hist, cur, poi_distance_matrix)


# speedup vs baseline: 3.4102x; 1.0213x over previous
import functools

import jax
import jax.numpy as jnp
from jax import lax
from jax.experimental import pallas as pl
from jax.experimental.pallas import tpu as pltpu
from jax.experimental.pallas import tpu_sc as plsc

P = 4096
SEQ = 4096
STATE = 2048
L = 16
CH = 16
U = 8
NVEC = SEQ // L
TC_BLK = 256


def _sc_body(hist_hbm, cur_hbm, dist_hbm, e_hbm,
             hist_v, cur_all_v, idx16_v, rows_v, ea_v, eb_v,
             sem_in, sem_a, sem_b):
    info = plsc.get_sparse_core_info()
    nc, ns = info.num_cores, info.num_subcores
    nw = nc * ns
    rows_per_w = STATE // nw
    wid = lax.axis_index("s") * nc + lax.axis_index("c")

    pltpu.sync_copy(hist_hbm, hist_v)
    pltpu.sync_copy(cur_hbm, cur_all_v)

    iota = lax.iota(jnp.int32, L)

    def gather_row(t, e_ref):
        c = t >> 4
        k = t - (c << 4)
        i = wid + t * nw
        kvec = jnp.full((L,), k, jnp.int32)

        @pl.when(k == 0)
        def _():
            rowidx = plsc.load_gather(
                cur_all_v, [wid + (c * CH + iota) * nw])
            idx16_v[pl.ds(0, L)] = rowidx
            pltpu.async_copy(dist_hbm.at[idx16_v], rows_v, sem_in).wait()

        nv2 = (((i + 1) >> 7) << 3) + 16

        @plsc.parallel_loop(0, nv2, unroll=U)
        def _(v):
            idx = hist_v[pl.ds(v * L, L)]
            g = plsc.load_gather(rows_v, [kvec, idx])
            e_ref[pl.ds(v * L, L)] = 1.0 / g

        return i

    def pair_body(q, carry):
        ia = gather_row(2 * q, ea_v)
        pltpu.async_copy(ea_v, e_hbm.at[ia], sem_a)

        @pl.when(q > 0)
        def _():
            pltpu.make_async_copy(eb_v, e_hbm.at[ia], sem_b).wait()

        ib = gather_row(2 * q + 1, eb_v)
        pltpu.async_copy(eb_v, e_hbm.at[ib], sem_b)
        pltpu.make_async_copy(ea_v, e_hbm.at[ia], sem_a).wait()
        return carry

    lax.fori_loop(0, rows_per_w // 2, pair_body, 0)
    pltpu.make_async_copy(eb_v, e_hbm.at[0], sem_b).wait()


_sc_energies = functools.partial(
    pl.kernel,
    out_type=jax.ShapeDtypeStruct((STATE, SEQ), jnp.float32),
    mesh=plsc.VectorSubcoreMesh(core_axis_name="c", subcore_axis_name="s"),
    compiler_params=pltpu.CompilerParams(
        use_tc_tiling_on_sc=False, needs_layout_passes=False),
    scratch_types=[
        pltpu.VMEM((SEQ,), jnp.int32),
        pltpu.VMEM((STATE,), jnp.int32),
        pltpu.VMEM((L,), jnp.int32),
        pltpu.VMEM((CH, SEQ), jnp.float32),
        pltpu.VMEM((SEQ,), jnp.float32),
        pltpu.VMEM((SEQ,), jnp.float32),
        pltpu.SemaphoreType.DMA,
        pltpu.SemaphoreType.DMA,
        pltpu.SemaphoreType.DMA,
    ],
)(_sc_body)


def _tc_softmax_body(e_ref, o_ref):
    b = pl.program_id(0)
    rows = jax.lax.broadcasted_iota(jnp.int32, (TC_BLK, SEQ), 0) + b * TC_BLK
    cols = jax.lax.broadcasted_iota(jnp.int32, (TC_BLK, SEQ), 1)
    e = jnp.where(cols <= rows, e_ref[...], 0.0)
    m = jnp.max(e, axis=1, keepdims=True)
    p = jnp.exp(e - m)
    s = jnp.sum(p, axis=1, keepdims=True)
    o_ref[...] = p / s


def _tc_softmax(e):
    return pl.pallas_call(
        _tc_softmax_body,
        grid=(STATE // TC_BLK,),
        in_specs=[pl.BlockSpec((TC_BLK, SEQ), lambda b: (b, 0))],
        out_specs=pl.BlockSpec((TC_BLK, SEQ), lambda b: (b, 0)),
        out_shape=jax.ShapeDtypeStruct((STATE, SEQ), jnp.float32),
    )(e)


def kernel(history, current, poi_distance_matrix):
    hist = history.astype(jnp.int32)
    cur = current.astype(jnp.int32)
    e = _sc_energies(hist, cur, poi_distance_matrix)
    return _tc_softmax(e)


# speedup vs baseline: 7.3103x; 2.1436x over previous
import functools

import jax
import jax.numpy as jnp
from jax import lax
from jax.experimental import pallas as pl
from jax.experimental.pallas import tpu as pltpu
from jax.experimental.pallas import tpu_sc as plsc

P = 4096
SEQ = 4096
STATE = 2048
L = 16
CH = 16
U = 8
NVEC = SEQ // L
TC_BLK = 256


def _sc_body(hist_hbm, cur_hbm, dist_hbm, e_hbm,
             hist_v, cur_all_v, idx16_v, rows_v, ea_v, eb_v,
             sem_in, sem_a, sem_b):
    info = plsc.get_sparse_core_info()
    nc, ns = info.num_cores, info.num_subcores
    nw = nc * ns
    rows_per_w = STATE // nw
    wid = lax.axis_index("s") * nc + lax.axis_index("c")

    pltpu.sync_copy(hist_hbm, hist_v)
    pltpu.sync_copy(cur_hbm, cur_all_v)

    iota = lax.iota(jnp.int32, L)

    def gather_row(t, e_ref):
        c = t >> 4
        k = t - (c << 4)
        i = wid + t * nw
        kvec = jnp.full((L,), k, jnp.int32)

        @pl.when(k == 0)
        def _():
            rowidx = plsc.load_gather(
                cur_all_v, [wid + (c * CH + iota) * nw])
            idx16_v[pl.ds(0, L)] = rowidx
            pltpu.async_copy(dist_hbm.at[idx16_v], rows_v, sem_in).wait()

        nv2 = (((i + 1) >> 7) << 3) + 16

        @plsc.parallel_loop(0, nv2, unroll=U)
        def _(v):
            idx = hist_v[pl.ds(v * L, L)]
            g = plsc.load_gather(rows_v, [kvec, idx])
            e_ref[pl.ds(v * L, L)] = 1.0 / g

        return i

    def pair_body(q, carry):
        ia = gather_row(2 * q, ea_v)
        pltpu.async_copy(ea_v, e_hbm.at[ia], sem_a)

        @pl.when(q > 0)
        def _():
            pltpu.make_async_copy(eb_v, e_hbm.at[ia], sem_b).wait()

        ib = gather_row(2 * q + 1, eb_v)
        pltpu.async_copy(eb_v, e_hbm.at[ib], sem_b)
        pltpu.make_async_copy(ea_v, e_hbm.at[ia], sem_a).wait()
        return carry

    lax.fori_loop(0, rows_per_w // 2, pair_body, 0)
    pltpu.make_async_copy(eb_v, e_hbm.at[0], sem_b).wait()


_sc_energies = functools.partial(
    pl.kernel,
    out_type=jax.ShapeDtypeStruct((STATE, SEQ), jnp.float32),
    mesh=plsc.VectorSubcoreMesh(core_axis_name="c", subcore_axis_name="s"),
    compiler_params=pltpu.CompilerParams(
        use_tc_tiling_on_sc=True, needs_layout_passes=False),
    scratch_types=[
        pltpu.VMEM((SEQ,), jnp.int32),
        pltpu.VMEM((STATE,), jnp.int32),
        pltpu.VMEM((L,), jnp.int32),
        pltpu.VMEM((CH, SEQ), jnp.float32),
        pltpu.VMEM((SEQ,), jnp.float32),
        pltpu.VMEM((SEQ,), jnp.float32),
        pltpu.SemaphoreType.DMA,
        pltpu.SemaphoreType.DMA,
        pltpu.SemaphoreType.DMA,
    ],
)(_sc_body)


def _tc_softmax_body(e_ref, o_ref):
    b = pl.program_id(0)
    rows = jax.lax.broadcasted_iota(jnp.int32, (TC_BLK, SEQ), 0) + b * TC_BLK
    cols = jax.lax.broadcasted_iota(jnp.int32, (TC_BLK, SEQ), 1)
    e = jnp.where(cols <= rows, e_ref[...], 0.0)
    m = jnp.max(e, axis=1, keepdims=True)
    p = jnp.exp(e - m)
    s = jnp.sum(p, axis=1, keepdims=True)
    o_ref[...] = p / s


def _tc_softmax(e):
    return pl.pallas_call(
        _tc_softmax_body,
        grid=(STATE // TC_BLK,),
        in_specs=[pl.BlockSpec((TC_BLK, SEQ), lambda b: (b, 0))],
        out_specs=pl.BlockSpec((TC_BLK, SEQ), lambda b: (b, 0)),
        out_shape=jax.ShapeDtypeStruct((STATE, SEQ), jnp.float32),
    )(e)


def kernel(history, current, poi_distance_matrix):
    hist = history.astype(jnp.int32)
    cur = current.astype(jnp.int32)
    e = _sc_energies(hist, cur, poi_distance_matrix)
    return _tc_softmax(e)
